# baseline (speedup 1.0000x reference)
import functools

import jax
import jax.numpy as jnp
from jax import lax
from jax.experimental import pallas as pl
from jax.experimental.pallas import tpu as pltpu
from jax.experimental.pallas import tpu_sc as plsc

N = 10000
E = 320000
D = 128

NC = 2
NS = 16
NW = NC * NS
EPW = E // NW
K = 80
EPWP = EPW
NCHUNK = EPWP // K
NP = 10240
RPS = NP // NS
RING = 4


def _spmm_body(groups, weighted, x_hbm, z_hbm, *rest):
    na = sum(len(g) for g in groups)
    off = 0
    if weighted:
        wv_hbm = rest[0]
        off = 1
    edge_refs = rest[off: off + 2 * na]
    out_hbm = rest[off + 2 * na]
    scr = rest[off + 2 * na + 1:]
    acc = scr[0]
    rowb = scr[1:1 + RING]
    cvb = scr[1 + RING:1 + 2 * RING]
    gath = scr[1 + 2 * RING:1 + 3 * RING]
    seme = scr[1 + 3 * RING:1 + 4 * RING]
    semg = scr[1 + 4 * RING:1 + 5 * RING]
    sems = scr[1 + 5 * RING:1 + 6 * RING]
    semz = scr[1 + 6 * RING]
    wbuf = scr[1 + 6 * RING + 1] if weighted else None

    c = lax.axis_index("c")
    s = lax.axis_index("s")
    wid = c * NS + s
    r0 = s * RPS

    if weighted:
        pltpu.sync_copy(wv_hbm, wbuf)
        w16 = wbuf[...]

    def start_idx(er, ecv, t, b):
        pltpu.async_copy(er.at[wid, t], rowb[b], seme[b])
        pltpu.async_copy(ecv.at[wid, t], cvb[b], seme[b])

    def wait_idx(er, ecv, t, b):
        pltpu.make_async_copy(er.at[wid, t], rowb[b], seme[b]).wait()
        pltpu.make_async_copy(ecv.at[wid, t], cvb[b], seme[b]).wait()

    def start_gather(b):
        pltpu.async_copy(x_hbm.at[cvb[b].at[pl.ds(0, K)]], gath[b], semg[b])

    def wait_gather(b):
        pltpu.make_async_copy(x_hbm.at[cvb[b].at[pl.ds(0, K)]], gath[b],
                              semg[b]).wait()

    def start_scatter(b):
        pltpu.async_copy(gath[b], acc.at[rowb[b]], sems[b], add=True)

    def wait_scatter(b):
        pltpu.make_async_copy(gath[b], acc.at[rowb[b]], sems[b]).wait()

    def make_scale(wsplat):
        def scale(b):
            def do16(base, v16, jlo):
                for j in range(jlo, 16):
                    k = base + j
                    bc = jnp.full((16,), v16[j], jnp.float32)
                    for d in range(D // 16):
                        sl = pl.ds(d * 16, 16)
                        gath[b][k, sl] = gath[b][k, sl] * bc

            def scale16(q, c2):
                v16 = lax.bitcast_convert_type(
                    cvb[b][pl.ds(K + q * 16, 16)], jnp.float32)
                if wsplat is not None:
                    v16 = v16 * wsplat
                do16(q * 16, v16, 0)
                return c2

            lax.fori_loop(0, K // 16, scale16, 0)

        return scale

    eidx = 0
    for gi, group in enumerate(groups):
        pltpu.async_copy(z_hbm.at[pl.ds(r0, RPS)], acc.at[pl.ds(r0, RPS)], semz)

        first = True
        for _ in group:
            er = edge_refs[2 * eidx]
            ecv = edge_refs[2 * eidx + 1]
            scale = make_scale(
                jnp.full((16,), w16[eidx], jnp.float32) if weighted else None)

            start_idx(er, ecv, 0, 0)
            start_idx(er, ecv, 1, 1)
            wait_idx(er, ecv, 0, 0)
            start_gather(0)
            if first:
                pltpu.make_async_copy(
                    z_hbm.at[pl.ds(r0, RPS)], acc.at[pl.ds(r0, RPS)],
                    semz).wait()
                plsc.subcore_barrier()
                first = False

            def step(t4, carry):
                for u in range(RING):
                    t = t4 * RING + u
                    b = u

                    @pl.when(jnp.logical_and(t >= 2, t < NCHUNK + 2))
                    def _():
                        wait_scatter((u + 2) % RING)

                    @pl.when(t + 2 < NCHUNK)
                    def _():
                        start_idx(er, ecv, t + 2, (u + 2) % RING)

                    @pl.when(t + 1 < NCHUNK)
                    def _():
                        wait_idx(er, ecv, t + 1, (u + 1) % RING)
                        start_gather((u + 1) % RING)

                    @pl.when(t < NCHUNK)
                    def _():
                        wait_gather(b)
                        scale(b)
                        start_scatter(b)
                return carry

            nsteps = (NCHUNK + 2 + RING - 1) // RING
            lax.fori_loop(0, nsteps, step, 0)
            eidx += 1

        plsc.subcore_barrier()
        pltpu.sync_copy(acc.at[pl.ds(r0, RPS)],
                        out_hbm.at[c, gi, pl.ds(r0, RPS)])
        plsc.subcore_barrier()


def _make_spmm(groups, weighted):
    na = sum(len(g) for g in groups)
    mesh = plsc.VectorSubcoreMesh(core_axis_name="c", subcore_axis_name="s")
    scratch = [pltpu.VMEM_SHARED((NP, D), jnp.float32)]
    scratch += [pltpu.VMEM((K,), jnp.int32) for _ in range(RING)]
    scratch += [pltpu.VMEM((2 * K,), jnp.int32) for _ in range(RING)]
    scratch += [pltpu.VMEM((K, D), jnp.float32) for _ in range(RING)]
    scratch += [pltpu.SemaphoreType.DMA for _ in range(3 * RING + 1)]
    if weighted:
        scratch += [pltpu.VMEM((16,), jnp.float32)]
    return pl.kernel(
        functools.partial(_spmm_body, groups, weighted),
        out_type=jax.ShapeDtypeStruct((NC, len(groups), NP, D), jnp.float32),
        mesh=mesh,
        scratch_types=scratch,
        name=f"sc_spmm_{len(groups)}_{na}",
    )


_spmm4 = _make_spmm(((0,), (1,), (2,), (3,)), False)
_spmm2w = _make_spmm(((0, 1),), True)

BN = 1000
BNP = 1024


def _mm_body(x_ref, w_ref, b_ref, o_ref):
    o_ref[...] = (
        jnp.dot(x_ref[...], w_ref[...], preferred_element_type=jnp.float32)
        + b_ref[...]
    )


def _matmul(x, w, b):
    return pl.pallas_call(
        _mm_body,
        grid=(N // BN,),
        in_specs=[
            pl.BlockSpec((BN, D), lambda n: (n, 0)),
            pl.BlockSpec((D, D), lambda n: (0, 0)),
            pl.BlockSpec((1, D), lambda n: (0, 0)),
        ],
        out_specs=pl.BlockSpec((BN, D), lambda n: (n, 0)),
        out_shape=jax.ShapeDtypeStruct((N, D), jnp.float32),
    )(x, w, b.reshape(1, D))


def _combo1_body(ws0_ref, wr0_ref, wrl0_ref, y_ref, s1_ref, yres_ref, yfin_ref):
    y = [y_ref[0, i] + y_ref[1, i] for i in range(4)]
    s1 = jnp.zeros((BNP, D), jnp.float32)
    for i in range(3):
        s1 += (ws0_ref[0, i] / 3.0) * y[i]
    s1_ref[...] = s1
    yres = jnp.zeros((BNP, D), jnp.float32)
    for i in range(4):
        yres += (wr0_ref[0, i] / 4.0) * y[i]
    yres_ref[...] = yres
    yfin = jnp.zeros((BNP, D), jnp.float32)
    for j, i in enumerate((0, 1, 3)):
        yfin += (wrl0_ref[0, j] / 3.0) * y[i]
    yfin_ref[...] = yfin


def _combo1(w_seq_0, w_res_0, w_res_last_0, yp):
    o = jax.ShapeDtypeStruct((NP, D), jnp.float32)
    return pl.pallas_call(
        _combo1_body,
        grid=(NP // BNP,),
        in_specs=[
            pl.BlockSpec(memory_space=pltpu.SMEM),
            pl.BlockSpec(memory_space=pltpu.SMEM),
            pl.BlockSpec(memory_space=pltpu.SMEM),
            pl.BlockSpec((NC, 4, BNP, D), lambda n: (0, 0, n, 0)),
        ],
        out_specs=[pl.BlockSpec((BNP, D), lambda n: (n, 0))] * 3,
        out_shape=[o, o, o],
    )(w_seq_0.reshape(1, 3), w_res_0.reshape(1, 4),
      w_res_last_0.reshape(1, 3), yp)


def _combo2_body(ws1_ref, wrl1_ref, z_ref, yres_ref, yfin_ref,
                 s2_ref, res_ref):
    z = [z_ref[0, i] + z_ref[1, i] for i in range(4)]
    s2 = yres_ref[...]
    for i in range(3):
        s2 += (ws1_ref[0, i] / 3.0) * z[i]
    s2_ref[...] = s2
    res = yfin_ref[...]
    for j, i in enumerate((0, 1, 3)):
        res += (wrl1_ref[0, j] / 3.0) * z[i]
    res_ref[...] = res


def _combo2(w_seq_1, w_res_last_1, zp, yres, yfin):
    o = jax.ShapeDtypeStruct((NP, D), jnp.float32)
    return pl.pallas_call(
        _combo2_body,
        grid=(NP // BNP,),
        in_specs=[
            pl.BlockSpec(memory_space=pltpu.SMEM),
            pl.BlockSpec(memory_space=pltpu.SMEM),
            pl.BlockSpec((NC, 4, BNP, D), lambda n: (0, 0, n, 0)),
            pl.BlockSpec((BNP, D), lambda n: (n, 0)),
            pl.BlockSpec((BNP, D), lambda n: (n, 0)),
        ],
        out_specs=[pl.BlockSpec((BNP, D), lambda n: (n, 0))] * 2,
        out_shape=[o, o],
    )(w_seq_1.reshape(1, 3), w_res_last_1.reshape(1, 3), zp, yres, yfin)


def _final_body(u_ref, res_ref, o_ref):
    acc = u_ref[0, 0] + u_ref[1, 0] + res_ref[...]
    mu = jnp.mean(acc, axis=-1, keepdims=True)
    ctr = acc - mu
    var = jnp.mean(ctr * ctr, axis=-1, keepdims=True)
    nrm = ctr * lax.rsqrt(var + 1e-5)
    o_ref[...] = 0.5 * nrm * (1.0 + lax.erf(nrm * (2.0 ** -0.5)))


def _final(up, res):
    return pl.pallas_call(
        _final_body,
        grid=(NP // BNP,),
        in_specs=[
            pl.BlockSpec((NC, 1, BNP, D), lambda n: (0, 0, n, 0)),
            pl.BlockSpec((BNP, D), lambda n: (n, 0)),
        ],
        out_specs=pl.BlockSpec((BNP, D), lambda n: (n, 0)),
        out_shape=jax.ShapeDtypeStruct((NP, D), jnp.float32),
    )(up, res)


@jax.jit
def kernel(x, affine_w, affine_b,
           adj0_rows, adj0_cols, adj0_vals,
           adj1_rows, adj1_cols, adj1_vals,
           adj2_rows, adj2_cols, adj2_vals,
           adj3_rows, adj3_cols, adj3_vals,
           w_seq_0, w_seq_1, w_seq_last, w_res_0, w_res_last_0, w_res_last_1):
    eshape = (NW, NCHUNK, K)
    npad = EPWP - EPW

    def pack(r, cc, v):
        if npad:
            r = jnp.concatenate(
                [r.reshape(NW, EPW),
                 jnp.full((NW, npad), N + 64, jnp.int32)], axis=1)
            cc = jnp.concatenate(
                [cc.reshape(NW, EPW), jnp.zeros((NW, npad), jnp.int32)],
                axis=1)
            v = jnp.concatenate(
                [v.reshape(NW, EPW), jnp.zeros((NW, npad), jnp.float32)],
                axis=1)
        cv = jnp.concatenate(
            [cc.reshape(eshape),
             lax.bitcast_convert_type(v, jnp.int32).reshape(eshape)], axis=2)
        return (r.reshape(eshape), cv)

    e0 = pack(adj0_rows, adj0_cols, adj0_vals)
    e1 = pack(adj1_rows, adj1_cols, adj1_vals)
    e2 = pack(adj2_rows, adj2_cols, adj2_vals)
    e3 = pack(adj3_rows, adj3_cols, adj3_vals)
    z = jnp.zeros((NP, D), jnp.float32)
    wv = jnp.zeros((16,), jnp.float32).at[0].set(w_seq_last[0] / 2.0)
    wv = wv.at[1].set(w_seq_last[1] / 2.0)

    h = _matmul(x, affine_w, affine_b)
    yp = _spmm4(h, z, *e0, *e1, *e2, *e3)
    s1, yres, yfin = _combo1(w_seq_0, w_res_0, w_res_last_0, yp)
    zp = _spmm4(s1, z, *e0, *e1, *e2, *e3)
    s2, res = _combo2(w_seq_1, w_res_last_1, zp, yres, yfin)
    up = _spmm2w(s2, z, wv, *e0, *e1)
    out = _final(up, res)
    return out[:N]


# speedup vs baseline: 1.0068x; 1.0068x over previous
import functools

import jax
import jax.numpy as jnp
from jax import lax
from jax.experimental import pallas as pl
from jax.experimental.pallas import tpu as pltpu
from jax.experimental.pallas import tpu_sc as plsc

N = 10000
E = 320000
D = 128

NC = 2
NS = 16
NW = NC * NS
EPW = E // NW
K = 80
EPWP = EPW
NCHUNK = EPWP // K
NP = 10240
RPS = NP // NS
RING = 4


def _spmm_body(groups, weighted, x_hbm, z_hbm, *rest):
    na = sum(len(g) for g in groups)
    off = 0
    if weighted:
        wv_hbm = rest[0]
        off = 1
    edge_refs = rest[off: off + 3 * na]
    out_hbm = rest[off + 3 * na]
    scr = rest[off + 3 * na + 1:]
    acc = scr[0]
    rowb = scr[1:1 + RING]
    colb = scr[1 + RING:1 + 2 * RING]
    valb = scr[1 + 2 * RING:1 + 3 * RING]
    gath = scr[1 + 3 * RING:1 + 4 * RING]
    seme = scr[1 + 4 * RING:1 + 5 * RING]
    semg = scr[1 + 5 * RING:1 + 6 * RING]
    sems = scr[1 + 6 * RING:1 + 7 * RING]
    semz = scr[1 + 7 * RING]
    wbuf = scr[1 + 7 * RING + 1] if weighted else None

    c = lax.axis_index("c")
    s = lax.axis_index("s")
    wid = c * NS + s
    r0 = s * RPS

    if weighted:
        pltpu.sync_copy(wv_hbm, wbuf)
        w16 = wbuf[...]

    def start_idx(er, ec, ev, t, b):
        pltpu.async_copy(er.at[wid, t], rowb[b], seme[b])
        pltpu.async_copy(ec.at[wid, t], colb[b], seme[b])
        pltpu.async_copy(ev.at[wid, t], valb[b], seme[b])

    def wait_idx(er, ec, ev, t, b):
        pltpu.make_async_copy(er.at[wid, t], rowb[b], seme[b]).wait()
        pltpu.make_async_copy(ec.at[wid, t], colb[b], seme[b]).wait()
        pltpu.make_async_copy(ev.at[wid, t], valb[b], seme[b]).wait()

    def start_gather(b):
        pltpu.async_copy(x_hbm.at[colb[b]], gath[b], semg[b])

    def wait_gather(b):
        pltpu.make_async_copy(x_hbm.at[colb[b]], gath[b], semg[b]).wait()

    def start_scatter(b):
        pltpu.async_copy(gath[b], acc.at[rowb[b]], sems[b], add=True)

    def wait_scatter(b):
        pltpu.make_async_copy(gath[b], acc.at[rowb[b]], sems[b]).wait()

    def make_scale(wsplat):
        def scale(b):
            def do16(base, v16, jlo):
                for j in range(jlo, 16):
                    k = base + j
                    bc = jnp.full((16,), v16[j], jnp.float32)
                    for d in range(D // 16):
                        sl = pl.ds(d * 16, 16)
                        gath[b][k, sl] = gath[b][k, sl] * bc

            def scale16(q, c2):
                v16 = valb[b][pl.ds(q * 16, 16)]
                if wsplat is not None:
                    v16 = v16 * wsplat
                do16(q * 16, v16, 0)
                return c2

            lax.fori_loop(0, K // 16, scale16, 0)
            if K % 16:
                v16 = valb[b][pl.ds(K - 16, 16)]
                if wsplat is not None:
                    v16 = v16 * wsplat
                do16(K - 16, v16, 16 - (K % 16))

        return scale

    eidx = 0
    for gi, group in enumerate(groups):
        pltpu.async_copy(z_hbm.at[pl.ds(r0, RPS)], acc.at[pl.ds(r0, RPS)], semz)

        first = True
        for _ in group:
            er = edge_refs[3 * eidx]
            ec = edge_refs[3 * eidx + 1]
            ev = edge_refs[3 * eidx + 2]
            scale = make_scale(
                jnp.full((16,), w16[eidx], jnp.float32) if weighted else None)

            start_idx(er, ec, ev, 0, 0)
            start_idx(er, ec, ev, 1, 1)
            wait_idx(er, ec, ev, 0, 0)
            start_gather(0)
            if first:
                pltpu.make_async_copy(
                    z_hbm.at[pl.ds(r0, RPS)], acc.at[pl.ds(r0, RPS)],
                    semz).wait()
                plsc.subcore_barrier()
                first = False

            def step(t4, carry):
                for u in range(RING):
                    t = t4 * RING + u
                    b = u

                    @pl.when(jnp.logical_and(t >= 2, t < NCHUNK + 2))
                    def _():
                        wait_scatter((u + 2) % RING)

                    @pl.when(t + 2 < NCHUNK)
                    def _():
                        start_idx(er, ec, ev, t + 2, (u + 2) % RING)

                    @pl.when(t + 1 < NCHUNK)
                    def _():
                        wait_idx(er, ec, ev, t + 1, (u + 1) % RING)
                        start_gather((u + 1) % RING)

                    @pl.when(t < NCHUNK)
                    def _():
                        wait_gather(b)
                        scale(b)
                        start_scatter(b)
                return carry

            nsteps = (NCHUNK + 2 + RING - 1) // RING
            lax.fori_loop(0, nsteps, step, 0)
            eidx += 1

        plsc.subcore_barrier()
        pltpu.sync_copy(acc.at[pl.ds(r0, RPS)],
                        out_hbm.at[c, gi, pl.ds(r0, RPS)])
        plsc.subcore_barrier()


def _make_spmm(groups, weighted):
    na = sum(len(g) for g in groups)
    mesh = plsc.VectorSubcoreMesh(core_axis_name="c", subcore_axis_name="s")
    scratch = [pltpu.VMEM_SHARED((NP, D), jnp.float32)]
    scratch += [pltpu.VMEM((K,), jnp.int32) for _ in range(RING)]
    scratch += [pltpu.VMEM((K,), jnp.int32) for _ in range(RING)]
    scratch += [pltpu.VMEM((K,), jnp.float32) for _ in range(RING)]
    scratch += [pltpu.VMEM((K, D), jnp.float32) for _ in range(RING)]
    scratch += [pltpu.SemaphoreType.DMA for _ in range(3 * RING + 1)]
    if weighted:
        scratch += [pltpu.VMEM((16,), jnp.float32)]
    return pl.kernel(
        functools.partial(_spmm_body, groups, weighted),
        out_type=jax.ShapeDtypeStruct((NC, len(groups), NP, D), jnp.float32),
        mesh=mesh,
        scratch_types=scratch,
        name=f"sc_spmm_{len(groups)}_{na}",
    )


_spmm4 = _make_spmm(((0,), (1,), (2,), (3,)), False)
_spmm2w = _make_spmm(((0, 1),), True)

BN = 1000
BNP = 1024


def _mm_body(x_ref, w_ref, b_ref, o_ref):
    o_ref[...] = (
        jnp.dot(x_ref[...], w_ref[...], preferred_element_type=jnp.float32)
        + b_ref[...]
    )


def _matmul(x, w, b):
    return pl.pallas_call(
        _mm_body,
        grid=(N // BN,),
        in_specs=[
            pl.BlockSpec((BN, D), lambda n: (n, 0)),
            pl.BlockSpec((D, D), lambda n: (0, 0)),
            pl.BlockSpec((1, D), lambda n: (0, 0)),
        ],
        out_specs=pl.BlockSpec((BN, D), lambda n: (n, 0)),
        out_shape=jax.ShapeDtypeStruct((N, D), jnp.float32),
    )(x, w, b.reshape(1, D))


def _combo1_body(ws0_ref, wr0_ref, wrl0_ref, y_ref, s1_ref, yres_ref, yfin_ref):
    y = [y_ref[0, i] + y_ref[1, i] for i in range(4)]
    s1 = jnp.zeros((BNP, D), jnp.float32)
    for i in range(3):
        s1 += (ws0_ref[0, i] / 3.0) * y[i]
    s1_ref[...] = s1
    yres = jnp.zeros((BNP, D), jnp.float32)
    for i in range(4):
        yres += (wr0_ref[0, i] / 4.0) * y[i]
    yres_ref[...] = yres
    yfin = jnp.zeros((BNP, D), jnp.float32)
    for j, i in enumerate((0, 1, 3)):
        yfin += (wrl0_ref[0, j] / 3.0) * y[i]
    yfin_ref[...] = yfin


def _combo1(w_seq_0, w_res_0, w_res_last_0, yp):
    o = jax.ShapeDtypeStruct((NP, D), jnp.float32)
    return pl.pallas_call(
        _combo1_body,
        grid=(NP // BNP,),
        in_specs=[
            pl.BlockSpec(memory_space=pltpu.SMEM),
            pl.BlockSpec(memory_space=pltpu.SMEM),
            pl.BlockSpec(memory_space=pltpu.SMEM),
            pl.BlockSpec((NC, 4, BNP, D), lambda n: (0, 0, n, 0)),
        ],
        out_specs=[pl.BlockSpec((BNP, D), lambda n: (n, 0))] * 3,
        out_shape=[o, o, o],
    )(w_seq_0.reshape(1, 3), w_res_0.reshape(1, 4),
      w_res_last_0.reshape(1, 3), yp)


def _combo2_body(ws1_ref, wrl1_ref, z_ref, yres_ref, yfin_ref,
                 s2_ref, res_ref):
    z = [z_ref[0, i] + z_ref[1, i] for i in range(4)]
    s2 = yres_ref[...]
    for i in range(3):
        s2 += (ws1_ref[0, i] / 3.0) * z[i]
    s2_ref[...] = s2
    res = yfin_ref[...]
    for j, i in enumerate((0, 1, 3)):
        res += (wrl1_ref[0, j] / 3.0) * z[i]
    res_ref[...] = res


def _combo2(w_seq_1, w_res_last_1, zp, yres, yfin):
    o = jax.ShapeDtypeStruct((NP, D), jnp.float32)
    return pl.pallas_call(
        _combo2_body,
        grid=(NP // BNP,),
        in_specs=[
            pl.BlockSpec(memory_space=pltpu.SMEM),
            pl.BlockSpec(memory_space=pltpu.SMEM),
            pl.BlockSpec((NC, 4, BNP, D), lambda n: (0, 0, n, 0)),
            pl.BlockSpec((BNP, D), lambda n: (n, 0)),
            pl.BlockSpec((BNP, D), lambda n: (n, 0)),
        ],
        out_specs=[pl.BlockSpec((BNP, D), lambda n: (n, 0))] * 2,
        out_shape=[o, o],
    )(w_seq_1.reshape(1, 3), w_res_last_1.reshape(1, 3), zp, yres, yfin)


def _final_body(u_ref, res_ref, o_ref):
    acc = u_ref[0, 0] + u_ref[1, 0] + res_ref[...]
    mu = jnp.mean(acc, axis=-1, keepdims=True)
    ctr = acc - mu
    var = jnp.mean(ctr * ctr, axis=-1, keepdims=True)
    nrm = ctr * lax.rsqrt(var + 1e-5)
    o_ref[...] = 0.5 * nrm * (1.0 + lax.erf(nrm * (2.0 ** -0.5)))


def _final(up, res):
    return pl.pallas_call(
        _final_body,
        grid=(NP // BNP,),
        in_specs=[
            pl.BlockSpec((NC, 1, BNP, D), lambda n: (0, 0, n, 0)),
            pl.BlockSpec((BNP, D), lambda n: (n, 0)),
        ],
        out_specs=pl.BlockSpec((BNP, D), lambda n: (n, 0)),
        out_shape=jax.ShapeDtypeStruct((NP, D), jnp.float32),
    )(up, res)


@jax.jit
def kernel(x, affine_w, affine_b,
           adj0_rows, adj0_cols, adj0_vals,
           adj1_rows, adj1_cols, adj1_vals,
           adj2_rows, adj2_cols, adj2_vals,
           adj3_rows, adj3_cols, adj3_vals,
           w_seq_0, w_seq_1, w_seq_last, w_res_0, w_res_last_0, w_res_last_1):
    eshape = (NW, NCHUNK, K)
    npad = EPWP - EPW

    def pack(r, cc, v):
        if npad:
            r = jnp.concatenate(
                [r.reshape(NW, EPW),
                 jnp.full((NW, npad), N + 64, jnp.int32)], axis=1)
            cc = jnp.concatenate(
                [cc.reshape(NW, EPW), jnp.zeros((NW, npad), jnp.int32)],
                axis=1)
            v = jnp.concatenate(
                [v.reshape(NW, EPW), jnp.zeros((NW, npad), jnp.float32)],
                axis=1)
        return (r.reshape(eshape), cc.reshape(eshape), v.reshape(eshape))

    e0 = pack(adj0_rows, adj0_cols, adj0_vals)
    e1 = pack(adj1_rows, adj1_cols, adj1_vals)
    e2 = pack(adj2_rows, adj2_cols, adj2_vals)
    e3 = pack(adj3_rows, adj3_cols, adj3_vals)
    z = jnp.zeros((NP, D), jnp.float32)
    wv = jnp.zeros((16,), jnp.float32).at[0].set(w_seq_last[0] / 2.0)
    wv = wv.at[1].set(w_seq_last[1] / 2.0)

    h = _matmul(x, affine_w, affine_b)
    yp = _spmm4(h, z, *e0, *e1, *e2, *e3)
    s1, yres, yfin = _combo1(w_seq_0, w_res_0, w_res_last_0, yp)
    zp = _spmm4(s1, z, *e0, *e1, *e2, *e3)
    s2, res = _combo2(w_seq_1, w_res_last_1, zp, yres, yfin)
    up = _spmm2w(s2, z, wv, *e0, *e1)
    out = _final(up, res)
    return out[:N]


# speedup vs baseline: 1.0228x; 1.0159x over previous
import functools

import jax
import jax.numpy as jnp
from jax import lax
from jax.experimental import pallas as pl
from jax.experimental.pallas import tpu as pltpu
from jax.experimental.pallas import tpu_sc as plsc

N = 10000
E = 320000
D = 128

NC = 2
NS = 16
NW = NC * NS
EPW = E // NW
K = 80
EPWP = EPW
NCHUNK = EPWP // K
NP = 10240
RPS = NP // NS
RING = 4


def _spmm_body(groups, weighted, x_hbm, z_hbm, *rest):
    na = sum(len(g) for g in groups)
    off = 0
    if weighted:
        wv_hbm = rest[0]
        off = 1
    edge_refs = rest[off: off + 3 * na]
    out_hbm = rest[off + 3 * na]
    scr = rest[off + 3 * na + 1:]
    acc = scr[0]
    rowb = scr[1:1 + RING]
    colb = scr[1 + RING:1 + 2 * RING]
    valb = scr[1 + 2 * RING:1 + 3 * RING]
    gath = scr[1 + 3 * RING:1 + 4 * RING]
    seme = scr[1 + 4 * RING:1 + 5 * RING]
    semg = scr[1 + 5 * RING:1 + 6 * RING]
    sems = scr[1 + 6 * RING:1 + 7 * RING]
    semz = scr[1 + 7 * RING]
    wbuf = scr[1 + 7 * RING + 1] if weighted else None

    c = lax.axis_index("c")
    s = lax.axis_index("s")
    wid = c * NS + s
    r0 = s * RPS

    if weighted:
        pltpu.sync_copy(wv_hbm, wbuf)
        w16 = wbuf[...]

    def start_idx(er, ec, ev, t, b):
        pltpu.async_copy(er.at[wid, t], rowb[b], seme[b])
        pltpu.async_copy(ec.at[wid, t], colb[b], seme[b])
        pltpu.async_copy(ev.at[wid, t], valb[b], seme[b])

    def wait_idx(er, ec, ev, t, b):
        pltpu.make_async_copy(er.at[wid, t], rowb[b], seme[b]).wait()
        pltpu.make_async_copy(ec.at[wid, t], colb[b], seme[b]).wait()
        pltpu.make_async_copy(ev.at[wid, t], valb[b], seme[b]).wait()

    def start_gather(b):
        pltpu.async_copy(x_hbm.at[colb[b]], gath[b], semg[b])

    def wait_gather(b):
        pltpu.make_async_copy(x_hbm.at[colb[b]], gath[b], semg[b]).wait()

    def start_scatter(b):
        pltpu.async_copy(gath[b], acc.at[rowb[b]], sems[b], add=True)

    def wait_scatter(b):
        pltpu.make_async_copy(gath[b], acc.at[rowb[b]], sems[b]).wait()

    def make_scale(wsplat):
        def scale(b):
            def do16(base, v16, jlo):
                for j in range(jlo, 16):
                    k = base + j
                    bc = jnp.full((16,), v16[j], jnp.float32)
                    for d in range(D // 16):
                        sl = pl.ds(d * 16, 16)
                        gath[b][k, sl] = gath[b][k, sl] * bc

            def scale16(q, c2):
                v16 = valb[b][pl.ds(q * 16, 16)]
                if wsplat is not None:
                    v16 = v16 * wsplat
                do16(q * 16, v16, 0)
                return c2

            lax.fori_loop(0, K // 16, scale16, 0)
            if K % 16:
                v16 = valb[b][pl.ds(K - 16, 16)]
                if wsplat is not None:
                    v16 = v16 * wsplat
                do16(K - 16, v16, 16 - (K % 16))

        return scale

    eidx = 0
    for gi, group in enumerate(groups):
        pltpu.async_copy(z_hbm.at[pl.ds(r0, RPS)], acc.at[pl.ds(r0, RPS)], semz)

        first = True
        for _ in group:
            er = edge_refs[3 * eidx]
            ec = edge_refs[3 * eidx + 1]
            ev = edge_refs[3 * eidx + 2]
            scale = make_scale(
                jnp.full((16,), w16[eidx], jnp.float32) if weighted else None)

            start_idx(er, ec, ev, 0, 0)
            start_idx(er, ec, ev, 1, 1)
            wait_idx(er, ec, ev, 0, 0)
            start_gather(0)
            if first:
                pltpu.make_async_copy(
                    z_hbm.at[pl.ds(r0, RPS)], acc.at[pl.ds(r0, RPS)],
                    semz).wait()
                plsc.subcore_barrier()
                first = False

            def step(t4, carry):
                for u in range(RING):
                    t = t4 * RING + u
                    b = u

                    @pl.when(jnp.logical_and(t >= 2, t < NCHUNK + 2))
                    def _():
                        wait_scatter((u + 2) % RING)

                    @pl.when(t + 2 < NCHUNK)
                    def _():
                        start_idx(er, ec, ev, t + 2, (u + 2) % RING)

                    @pl.when(t + 1 < NCHUNK)
                    def _():
                        wait_idx(er, ec, ev, t + 1, (u + 1) % RING)
                        start_gather((u + 1) % RING)

                    @pl.when(t < NCHUNK)
                    def _():
                        wait_gather(b)
                        scale(b)
                        start_scatter(b)
                return carry

            nsteps = (NCHUNK + 2 + RING - 1) // RING
            lax.fori_loop(0, nsteps, step, 0)
            eidx += 1

        plsc.subcore_barrier()
        pltpu.sync_copy(acc.at[pl.ds(r0, RPS)],
                        out_hbm.at[c, gi, pl.ds(r0, RPS)])
        plsc.subcore_barrier()


def _make_spmm(groups, weighted):
    na = sum(len(g) for g in groups)
    mesh = plsc.VectorSubcoreMesh(core_axis_name="c", subcore_axis_name="s")
    scratch = [pltpu.VMEM_SHARED((NP, D), jnp.float32)]
    scratch += [pltpu.VMEM((K,), jnp.int32) for _ in range(RING)]
    scratch += [pltpu.VMEM((K,), jnp.int32) for _ in range(RING)]
    scratch += [pltpu.VMEM((K,), jnp.float32) for _ in range(RING)]
    scratch += [pltpu.VMEM((K, D), jnp.float32) for _ in range(RING)]
    scratch += [pltpu.SemaphoreType.DMA for _ in range(3 * RING + 1)]
    if weighted:
        scratch += [pltpu.VMEM((16,), jnp.float32)]
    return pl.kernel(
        functools.partial(_spmm_body, groups, weighted),
        out_type=jax.ShapeDtypeStruct((NC, len(groups), NP, D), jnp.float32),
        mesh=mesh,
        scratch_types=scratch,
        name=f"sc_spmm_{len(groups)}_{na}",
    )


_spmm3 = _make_spmm(((0,), (1,), (2,)), False)
_spmm1 = _make_spmm(((0,),), False)
_spmm2w = _make_spmm(((0, 1),), True)

BN = 1000
BNP = 1024


def _mm_body(x_ref, w_ref, b_ref, o_ref):
    o_ref[...] = (
        jnp.dot(x_ref[...], w_ref[...], preferred_element_type=jnp.float32)
        + b_ref[...]
    )


def _matmul(x, w, b):
    return pl.pallas_call(
        _mm_body,
        grid=(N // BN,),
        in_specs=[
            pl.BlockSpec((BN, D), lambda n: (n, 0)),
            pl.BlockSpec((D, D), lambda n: (0, 0)),
            pl.BlockSpec((1, D), lambda n: (0, 0)),
        ],
        out_specs=pl.BlockSpec((BN, D), lambda n: (n, 0)),
        out_shape=jax.ShapeDtypeStruct((N, D), jnp.float32),
    )(x, w, b.reshape(1, D))


def _combo_s1_body(ws0_ref, y_ref, s1_ref):
    s1 = jnp.zeros((BNP, D), jnp.float32)
    for i in range(3):
        s1 += (ws0_ref[0, i] / 3.0) * (y_ref[0, i] + y_ref[1, i])
    s1_ref[...] = s1


def _combo_s1(w_seq_0, yp3):
    return pl.pallas_call(
        _combo_s1_body,
        grid=(NP // BNP,),
        in_specs=[
            pl.BlockSpec(memory_space=pltpu.SMEM),
            pl.BlockSpec((NC, 3, BNP, D), lambda n: (0, 0, n, 0)),
        ],
        out_specs=pl.BlockSpec((BNP, D), lambda n: (n, 0)),
        out_shape=jax.ShapeDtypeStruct((NP, D), jnp.float32),
    )(w_seq_0.reshape(1, 3), yp3)


def _combo_res1_body(wr0_ref, wrl0_ref, y_ref, y3_ref, yres_ref, yfin_ref):
    y = [y_ref[0, i] + y_ref[1, i] for i in range(3)]
    y.append(y3_ref[0, 0] + y3_ref[1, 0])
    yres = jnp.zeros((BNP, D), jnp.float32)
    for i in range(4):
        yres += (wr0_ref[0, i] / 4.0) * y[i]
    yres_ref[...] = yres
    yfin = jnp.zeros((BNP, D), jnp.float32)
    for j, i in enumerate((0, 1, 3)):
        yfin += (wrl0_ref[0, j] / 3.0) * y[i]
    yfin_ref[...] = yfin


def _combo_res1(w_res_0, w_res_last_0, yp3, yp1):
    o = jax.ShapeDtypeStruct((NP, D), jnp.float32)
    return pl.pallas_call(
        _combo_res1_body,
        grid=(NP // BNP,),
        in_specs=[
            pl.BlockSpec(memory_space=pltpu.SMEM),
            pl.BlockSpec(memory_space=pltpu.SMEM),
            pl.BlockSpec((NC, 3, BNP, D), lambda n: (0, 0, n, 0)),
            pl.BlockSpec((NC, 1, BNP, D), lambda n: (0, 0, n, 0)),
        ],
        out_specs=[pl.BlockSpec((BNP, D), lambda n: (n, 0))] * 2,
        out_shape=[o, o],
    )(w_res_0.reshape(1, 4), w_res_last_0.reshape(1, 3), yp3, yp1)


def _combo_s2_body(ws1_ref, z_ref, yres_ref, s2_ref):
    s2 = yres_ref[...]
    for i in range(3):
        s2 += (ws1_ref[0, i] / 3.0) * (z_ref[0, i] + z_ref[1, i])
    s2_ref[...] = s2


def _combo_s2(w_seq_1, zp3, yres):
    return pl.pallas_call(
        _combo_s2_body,
        grid=(NP // BNP,),
        in_specs=[
            pl.BlockSpec(memory_space=pltpu.SMEM),
            pl.BlockSpec((NC, 3, BNP, D), lambda n: (0, 0, n, 0)),
            pl.BlockSpec((BNP, D), lambda n: (n, 0)),
        ],
        out_specs=pl.BlockSpec((BNP, D), lambda n: (n, 0)),
        out_shape=jax.ShapeDtypeStruct((NP, D), jnp.float32),
    )(w_seq_1.reshape(1, 3), zp3, yres)


def _combo_res2_body(wrl1_ref, z_ref, z3_ref, yfin_ref, res_ref):
    z = {0: z_ref[0, 0] + z_ref[1, 0], 1: z_ref[0, 1] + z_ref[1, 1],
         3: z3_ref[0, 0] + z3_ref[1, 0]}
    res = yfin_ref[...]
    for j, i in enumerate((0, 1, 3)):
        res += (wrl1_ref[0, j] / 3.0) * z[i]
    res_ref[...] = res


def _combo_res2(w_res_last_1, zp3, zp1, yfin):
    return pl.pallas_call(
        _combo_res2_body,
        grid=(NP // BNP,),
        in_specs=[
            pl.BlockSpec(memory_space=pltpu.SMEM),
            pl.BlockSpec((NC, 3, BNP, D), lambda n: (0, 0, n, 0)),
            pl.BlockSpec((NC, 1, BNP, D), lambda n: (0, 0, n, 0)),
            pl.BlockSpec((BNP, D), lambda n: (n, 0)),
        ],
        out_specs=pl.BlockSpec((BNP, D), lambda n: (n, 0)),
        out_shape=jax.ShapeDtypeStruct((NP, D), jnp.float32),
    )(w_res_last_1.reshape(1, 3), zp3, zp1, yfin)


def _final_body(u_ref, res_ref, o_ref):
    acc = u_ref[0, 0] + u_ref[1, 0] + res_ref[...]
    mu = jnp.mean(acc, axis=-1, keepdims=True)
    ctr = acc - mu
    var = jnp.mean(ctr * ctr, axis=-1, keepdims=True)
    nrm = ctr * lax.rsqrt(var + 1e-5)
    o_ref[...] = 0.5 * nrm * (1.0 + lax.erf(nrm * (2.0 ** -0.5)))


def _final(up, res):
    return pl.pallas_call(
        _final_body,
        grid=(NP // BNP,),
        in_specs=[
            pl.BlockSpec((NC, 1, BNP, D), lambda n: (0, 0, n, 0)),
            pl.BlockSpec((BNP, D), lambda n: (n, 0)),
        ],
        out_specs=pl.BlockSpec((BNP, D), lambda n: (n, 0)),
        out_shape=jax.ShapeDtypeStruct((NP, D), jnp.float32),
    )(up, res)


@jax.jit
def kernel(x, affine_w, affine_b,
           adj0_rows, adj0_cols, adj0_vals,
           adj1_rows, adj1_cols, adj1_vals,
           adj2_rows, adj2_cols, adj2_vals,
           adj3_rows, adj3_cols, adj3_vals,
           w_seq_0, w_seq_1, w_seq_last, w_res_0, w_res_last_0, w_res_last_1):
    eshape = (NW, NCHUNK, K)
    npad = EPWP - EPW

    def pack(r, cc, v):
        if npad:
            r = jnp.concatenate(
                [r.reshape(NW, EPW),
                 jnp.full((NW, npad), N + 64, jnp.int32)], axis=1)
            cc = jnp.concatenate(
                [cc.reshape(NW, EPW), jnp.zeros((NW, npad), jnp.int32)],
                axis=1)
            v = jnp.concatenate(
                [v.reshape(NW, EPW), jnp.zeros((NW, npad), jnp.float32)],
                axis=1)
        return (r.reshape(eshape), cc.reshape(eshape), v.reshape(eshape))

    e0 = pack(adj0_rows, adj0_cols, adj0_vals)
    e1 = pack(adj1_rows, adj1_cols, adj1_vals)
    e2 = pack(adj2_rows, adj2_cols, adj2_vals)
    e3 = pack(adj3_rows, adj3_cols, adj3_vals)
    z = jnp.zeros((NP, D), jnp.float32)
    wv = jnp.zeros((16,), jnp.float32).at[0].set(w_seq_last[0] / 2.0)
    wv = wv.at[1].set(w_seq_last[1] / 2.0)

    h = _matmul(x, affine_w, affine_b)
    yp3 = _spmm3(h, z, *e0, *e1, *e2)
    yp1 = _spmm1(h, z, *e3)
    s1 = _combo_s1(w_seq_0, yp3)
    zp3 = _spmm3(s1, z, *e0, *e1, *e2)
    zp1 = _spmm1(s1, z, *e3)
    yres, yfin = _combo_res1(w_res_0, w_res_last_0, yp3, yp1)
    s2 = _combo_s2(w_seq_1, zp3, yres)
    up = _spmm2w(s2, z, wv, *e0, *e1)
    res = _combo_res2(w_res_last_1, zp3, zp1, yfin)
    out = _final(up, res)
    return out[:N]
